# baseline (speedup 1.0000x reference)
import functools

import jax
import jax.numpy as jnp
from jax import lax
from jax.experimental import pallas as pl
from jax.experimental.pallas import tpu as pltpu
from jax.experimental.pallas import tpu_sc as plsc

N = 10000
E = 320000
D = 128
NC = 2
NS = 16
CH = 128
NCH = 160
EPT = NCH * CH
EP = NS * EPT
GCH = 8
NGROUP = NCH // GCH
AGGR = 2504
HALF0 = 5008
SENT = -1


def _segsum_impl(h_hbm, src_hbm, dst_hbm, zrow_hbm, agg_out, cnt_out,
                 sidx, didx, rows, h_sh, agg_sh, cnt_sh, ones_v, zbuf,
                 wbuf, gsem):
    with_cnt = cnt_out is not None
    c = lax.axis_index("c")
    s = lax.axis_index("s")
    size = 2504 - 8 * c

    @pl.when(s < 10)
    def _stage_h():
        pltpu.sync_copy(h_hbm.at[pl.ds(s * 1000, 1000)],
                        h_sh.at[pl.ds(s * 1000, 1000)])

    if with_cnt:
        def _ones_body(i, _):
            ones_v[pl.ds(i * 16, 16)] = jnp.full((16,), 1.0, jnp.float32)
            return 0
        lax.fori_loop(0, CH // 16, _ones_body, 0)

        def _zb_body(i, _):
            zbuf[pl.ds(i * 16, 16)] = jnp.zeros((16,), jnp.float32)
            return 0
        lax.fori_loop(0, 63, _zb_body, 0)

    for p in range(2):
        start = pl.multiple_of(c * HALF0 + p * size, 8)

        @pl.when(s < 2)
        def _z01():
            pltpu.sync_copy(zrow_hbm.at[pl.ds(s * 1000, 1000)],
                            agg_sh.at[pl.ds(s * 1000, 1000)])

        @pl.when(s == 2)
        def _z2():
            pltpu.sync_copy(zrow_hbm.at[pl.ds(2000, 504)],
                            agg_sh.at[pl.ds(2000, 504)])

        if with_cnt:
            @pl.when((s == 3) | (s == 4))
            def _zc():
                pltpu.sync_copy(zbuf.at[pl.ds(0, 1000)],
                                cnt_sh.at[pl.ds((s - 3) * 1000, 1000)])

            @pl.when(s == 5)
            def _zc2():
                pltpu.sync_copy(zbuf.at[pl.ds(0, 504)],
                                cnt_sh.at[pl.ds(2000, 504)])

        plsc.subcore_barrier()

        for g in range(NGROUP):
            pltpu.sync_copy(src_hbm.at[s, pl.ds(g * GCH, GCH)], sidx)
            pltpu.sync_copy(dst_hbm.at[s, pl.ds(g * GCH, GCH)], didx)

            def _tx(j, _):
                r = j // 8
                l = (j % 8) * 16
                dv = didx[r, pl.ds(l, 16)]
                sv = sidx[r, pl.ds(l, 16)]
                inr = (dv >= start) & (dv < start + size)
                didx[r, pl.ds(l, 16)] = jnp.where(inr, dv - start, SENT)
                sidx[r, pl.ds(l, 16)] = jnp.where(inr, sv, SENT)
                return 0
            lax.fori_loop(0, GCH * 8, _tx, 0)

            def _chunk(k, _):
                pltpu.async_copy(
                    h_sh.at[plsc.Indices(sidx.at[k], ignored_value=SENT)],
                    rows, gsem).wait()
                pltpu.sync_copy(
                    rows,
                    agg_sh.at[plsc.Indices(didx.at[k], ignored_value=SENT)],
                    add=True)
                if with_cnt:
                    pltpu.sync_copy(
                        ones_v,
                        cnt_sh.at[plsc.Indices(didx.at[k],
                                               ignored_value=SENT)],
                        add=True)
                return 0
            lax.fori_loop(0, GCH, _chunk, 0)

        plsc.subcore_barrier()

        @pl.when(s < 2)
        def _w01():
            pltpu.sync_copy(agg_sh.at[pl.ds(s * 1000, 1000)],
                            agg_out.at[pl.ds(start + s * 1000, 1000)])

        @pl.when((s == 2) & (c == 0))
        def _w2a():
            pltpu.sync_copy(agg_sh.at[pl.ds(2000, 504)],
                            agg_out.at[pl.ds(start + 2000, 504)])

        @pl.when((s == 2) & (c == 1))
        def _w2b():
            pltpu.sync_copy(agg_sh.at[pl.ds(2000, 496)],
                            agg_out.at[pl.ds(start + 2000, 496)])

        if with_cnt:
            @pl.when((s == 3) | (s == 4))
            def _wc():
                r0 = (s - 3) * 1000
                pltpu.sync_copy(cnt_sh.at[pl.ds(r0, 1000)],
                                wbuf.at[pl.ds(0, 1000)])
                pltpu.sync_copy(wbuf.at[pl.ds(0, 1000)],
                                cnt_out.at[pl.ds(start + r0, 1000)])

            @pl.when((s == 5) & (c == 0))
            def _wca():
                pltpu.sync_copy(cnt_sh.at[pl.ds(2000, 504)],
                                wbuf.at[pl.ds(0, 504)])
                pltpu.sync_copy(wbuf.at[pl.ds(0, 504)],
                                cnt_out.at[pl.ds(start + 2000, 504)])

            @pl.when((s == 5) & (c == 1))
            def _wcb():
                pltpu.sync_copy(cnt_sh.at[pl.ds(2000, 496)],
                                wbuf.at[pl.ds(0, 496)])
                pltpu.sync_copy(wbuf.at[pl.ds(0, 496)],
                                cnt_out.at[pl.ds(start + 2000, 496)])


def _segsum_body_cnt(h_hbm, src_hbm, dst_hbm, zrow_hbm, agg_out, cnt_out,
                     sidx, didx, rows, ones_v, zbuf, wbuf, h_sh, agg_sh,
                     cnt_sh, gsem):
    _segsum_impl(h_hbm, src_hbm, dst_hbm, zrow_hbm, agg_out, cnt_out,
                 sidx, didx, rows, h_sh, agg_sh, cnt_sh, ones_v, zbuf,
                 wbuf, gsem)


def _segsum_body_nocnt(h_hbm, src_hbm, dst_hbm, zrow_hbm, agg_out,
                       sidx, didx, rows, h_sh, agg_sh, gsem):
    _segsum_impl(h_hbm, src_hbm, dst_hbm, zrow_hbm, agg_out, None,
                 sidx, didx, rows, h_sh, agg_sh, None, None, None, None,
                 gsem)


@functools.lru_cache(maxsize=None)
def _make_segsum(with_cnt):
    mesh = plsc.VectorSubcoreMesh(
        core_axis_name="c", subcore_axis_name="s",
        num_cores=NC, num_subcores=NS)
    agg_t = jax.ShapeDtypeStruct((N, D), jnp.float32)
    cnt_t = jax.ShapeDtypeStruct((N,), jnp.float32)
    common = [
        pltpu.VMEM((GCH, CH), jnp.int32),
        pltpu.VMEM((GCH, CH), jnp.int32),
        pltpu.VMEM((CH, D), jnp.float32),
    ]
    if with_cnt:
        return pl.kernel(
            _segsum_body_cnt,
            out_type=(agg_t, cnt_t),
            mesh=mesh,
            scratch_types=common + [
                pltpu.VMEM((CH,), jnp.float32),
                pltpu.VMEM((1008,), jnp.float32),
                pltpu.VMEM((1008,), jnp.float32),
                pltpu.VMEM_SHARED((N, D), jnp.float32),
                pltpu.VMEM_SHARED((AGGR, D), jnp.float32),
                pltpu.VMEM_SHARED((AGGR,), jnp.float32),
                pltpu.SemaphoreType.DMA,
            ],
        )
    return pl.kernel(
        _segsum_body_nocnt,
        out_type=agg_t,
        mesh=mesh,
        scratch_types=common + [
            pltpu.VMEM_SHARED((N, D), jnp.float32),
            pltpu.VMEM_SHARED((AGGR, D), jnp.float32),
            pltpu.SemaphoreType.DMA,
        ],
    )


R = 1000
GRID = N // R


def _layer_mm_body(aref, cref, href, wl_ref, b_ref, wr_ref,
                   oref, sref, qref):
    i = pl.program_id(0)
    cnt = jnp.maximum(cref[...], 1.0)
    mean = aref[...] / cnt
    hp = (jnp.dot(mean, wl_ref[...], preferred_element_type=jnp.float32)
          + b_ref[...]
          + jnp.dot(href[...], wr_ref[...], preferred_element_type=jnp.float32))
    oref[...] = hp

    @pl.when(i == 0)
    def _init():
        sref[...] = jnp.zeros_like(sref)
        qref[...] = jnp.zeros_like(qref)

    sref[...] += jnp.sum(hp, axis=0, keepdims=True)
    qref[...] += jnp.sum(hp * hp, axis=0, keepdims=True)


def _final_mm_body(aref, cref, href, wl_ref, b_ref, wr_ref, oref):
    cnt = jnp.maximum(cref[...], 1.0)
    mean = aref[...] / cnt
    oref[...] = (jnp.dot(mean, wl_ref[...], preferred_element_type=jnp.float32)
                 + b_ref[...]
                 + jnp.dot(href[...], wr_ref[...],
                           preferred_element_type=jnp.float32))


def _norm_relu_body(href, sref, qref, gref, bref, oref):
    m = sref[...] / float(N)
    v = qref[...] / float(N) - m * m
    scale = gref[...] * lax.rsqrt(v + 1e-5)
    oref[...] = jnp.maximum((href[...] - m) * scale + bref[...], 0.0)


_row_spec = pl.BlockSpec((R, D), lambda i: (i, 0))
_cnt_spec = pl.BlockSpec((R, 1), lambda i: (i, 0))
_w_spec = pl.BlockSpec((D, D), lambda i: (0, 0))
_vec_spec = pl.BlockSpec((1, D), lambda i: (0, 0))

_layer_mm = pl.pallas_call(
    _layer_mm_body,
    grid=(GRID,),
    in_specs=[_row_spec, _cnt_spec, _row_spec, _w_spec, _vec_spec, _w_spec],
    out_specs=[_row_spec, _vec_spec, _vec_spec],
    out_shape=[
        jax.ShapeDtypeStruct((N, D), jnp.float32),
        jax.ShapeDtypeStruct((1, D), jnp.float32),
        jax.ShapeDtypeStruct((1, D), jnp.float32),
    ],
)

_final_mm = pl.pallas_call(
    _final_mm_body,
    grid=(GRID,),
    in_specs=[_row_spec, _cnt_spec, _row_spec, _w_spec, _vec_spec, _w_spec],
    out_specs=_row_spec,
    out_shape=jax.ShapeDtypeStruct((N, D), jnp.float32),
)

_norm_relu = pl.pallas_call(
    _norm_relu_body,
    grid=(GRID,),
    in_specs=[_row_spec, _vec_spec, _vec_spec, _vec_spec, _vec_spec],
    out_specs=_row_spec,
    out_shape=jax.ShapeDtypeStruct((N, D), jnp.float32),
)


def kernel(x, edge_index, Wl, bl, Wr, gamma, beta):
    npad_e = EP - E
    src = jnp.concatenate(
        [edge_index[0], jnp.zeros((npad_e,), jnp.int32)]).reshape(NS, NCH, CH)
    dst = jnp.concatenate(
        [edge_index[1],
         jnp.full((npad_e,), N, jnp.int32)]).reshape(NS, NCH, CH)
    zrow = jnp.zeros((AGGR, D), jnp.float32)
    WlT = jnp.swapaxes(Wl, 1, 2)
    WrT = jnp.swapaxes(Wr, 1, 2)
    bl2 = bl.reshape(Wl.shape[0], 1, D)
    gamma2 = gamma.reshape(-1, 1, D)
    beta2 = beta.reshape(-1, 1, D)

    h = x
    cnt2 = None
    for i in range(2):
        if i == 0:
            agg, cnt = _make_segsum(True)(h, src, dst, zrow)
            cnt2 = cnt.reshape(N, 1)
        else:
            agg = _make_segsum(False)(h, src, dst, zrow)
        hpre, ssum, ssq = _layer_mm(agg, cnt2, h, WlT[i], bl2[i], WrT[i])
        h = _norm_relu(hpre, ssum, ssq, gamma2[i], beta2[i])
    agg = _make_segsum(False)(h, src, dst, zrow)
    return _final_mm(agg, cnt2, h, WlT[2], bl2[2], WrT[2])
